# baseline (speedup 1.0000x reference)
import dataclasses
import functools

import jax
import jax.numpy as jnp
from jax import lax
from jax.experimental import pallas as pl
from jax.experimental.pallas import tpu as pltpu
from jax.experimental.pallas import tpu_sc as plsc

SEC_PER_A = 31556926.0
RHO_W = 1000.0
RHO_I = 917.0
GRAVITY = 9.81
FLOW_COEFF = 0.0405
FLOW_EXP = 1.25
N_FLOW_ITERS = 10

NN = 100000
OWN = 3136
NPAD = 32 * OWN
EE = 1600000
NWORK = 32
ROWS_PER_TILE = 400
EROWS = NWORK * ROWS_PER_TILE
EPAD = EROWS * 128
CH_ROWS = 16
CHUNK = CH_ROWS * 128
N_CHUNKS = ROWS_PER_TILE // CH_ROWS
EALLOC = EPAD + NWORK * CHUNK

_SC_MESH = plsc.VectorSubcoreMesh(core_axis_name="c", subcore_axis_name="s")

_SC_PARAMS = pltpu.CompilerParams()
if "needs_layout_passes" in pltpu.CompilerParams.__dataclass_fields__:
    _SC_PARAMS = dataclasses.replace(_SC_PARAMS, needs_layout_passes=False)


def _f32(shape):
    return jax.ShapeDtypeStruct(shape, jnp.float32)


def _i32(shape):
    return jax.ShapeDtypeStruct(shape, jnp.int32)


def _owner16(d16):
    q = lax.shift_right_logical(d16, 6)
    return lax.shift_right_logical(q * 1338, 16)


def _pick(vecs, idx):
    r = jnp.int32(-2147483647)
    for h, v in enumerate(vecs):
        lane = idx - 16 * h
        m = lax.iota(jnp.int32, 16) == lane
        x = lax.reduce_max(jnp.where(m, v, jnp.int32(-2147483647)), axes=(0,))
        r = jnp.maximum(r, x)
    return r


def _tc_elementwise(body, n_out, *arrays):
    rows = NPAD // 128
    outs = pl.pallas_call(
        body,
        out_shape=[jax.ShapeDtypeStruct((rows, 128), jnp.float32)] * n_out,
    )(*[a.reshape(rows, 128) for a in arrays])
    return [o.reshape(NPAD) for o in outs]


def _pre_body(mr_ref, ca_ref, be_ref, wp_ref, ss_ref, bs_ref,
              melt_ref, pot_ref, geo_ref):
    melt_ref[...] = mr_ref[...] * ca_ref[...] * (1.0 / SEC_PER_A)
    pot_ref[...] = RHO_W * GRAVITY * be_ref[...] + wp_ref[...]
    geo_ref[...] = (-RHO_I * GRAVITY * ss_ref[...]
                    - (RHO_W - RHO_I) * GRAVITY * bs_ref[...])


def _g0_body(melt_ref, ws0_ref, ws1_ref, g_ref, wst_ref):
    wst = ws0_ref[...] + ws1_ref[...]
    wst_ref[...] = wst
    g_ref[...] = melt_ref[...] / (wst + 1e-12)


def _fin_body(dis_ref, ca_ref, st_ref, hg_ref):
    grad = (dis_ref[...] * FLOW_COEFF * ca_ref[...] ** FLOW_EXP) ** 2
    hg_ref[...] = jnp.where(st_ref[...] == 0.0, grad, 0.0)


def _hist_kernel(dst_hbm, counts_hbm, dbuf, cnt_v, sem):
    cid = lax.axis_index("c")
    sid = lax.axis_index("s")
    wid = sid * 2 + cid
    base_row = pl.multiple_of(wid * ROWS_PER_TILE, 16)

    for h in range(2):
        cnt_v[pl.ds(h * 16, 16)] = jnp.zeros((16,), jnp.int32)

    @pl.loop(0, N_CHUNKS)
    def _(ci):
        pltpu.sync_copy(dst_hbm.at[pl.ds(base_row + ci * CH_ROWS, CH_ROWS)],
                        dbuf)
        for j in range(CH_ROWS):
            @pl.loop(0, 128, step=16)
            def _(k):
                o16 = _owner16(dbuf[j, pl.ds(k, 16)])
                c16, m16 = plsc.scan_count(o16)
                plsc.addupdate_scatter(cnt_v, [o16], c16, mask=m16)

    pltpu.sync_copy(
        cnt_v, counts_hbm.at[pl.ds(pl.multiple_of(wid * NWORK, 32), NWORK)])


def _b2_kernel(src_hbm, dst_hbm, counts_hbm,
               bpk_hbm, offs_hbm, tots_hbm,
               sbuf, dbuf, pkbuf, posbuf, cnts_v, postbl, aux_v, sem):
    cid = lax.axis_index("c")
    sid = lax.axis_index("s")
    wid = sid * 2 + cid
    base_row = pl.multiple_of(wid * ROWS_PER_TILE, 16)

    pltpu.sync_copy(counts_hbm, cnts_v)

    tot0 = jnp.zeros((16,), jnp.int32)
    tot1 = jnp.zeros((16,), jnp.int32)
    pri0 = jnp.zeros((16,), jnp.int32)
    pri1 = jnp.zeros((16,), jnp.int32)
    for p in range(NWORK):
        r0 = cnts_v[pl.ds(p * NWORK, 16)]
        r1 = cnts_v[pl.ds(p * NWORK + 16, 16)]
        tot0 = tot0 + r0
        tot1 = tot1 + r1
        use = jnp.where(jnp.int32(p) < wid, jnp.int32(1), jnp.int32(0))
        pri0 = pri0 + r0 * use
        pri1 = pri1 + r1 * use

    ru0 = lax.shift_left(
        lax.shift_right_logical(tot0 + (CHUNK - 1), 11), 11)
    ru1 = lax.shift_left(
        lax.shift_right_logical(tot1 + (CHUNK - 1), 11), 11)
    c0 = plsc.cumsum(ru0)
    c1 = plsc.cumsum(ru1)
    half = lax.reduce_max(c0, axes=(0,))
    off0 = c0 - ru0
    off1 = c1 - ru1 + half
    endv = lax.reduce_max(c1, axes=(0,)) + half

    aux_v[pl.ds(0, 16)] = off0
    aux_v[pl.ds(16, 16)] = off1
    aux_v[pl.ds(32, 16)] = jnp.full((16,), 1, jnp.int32) * endv
    aux_v[pl.ds(48, 16)] = tot0
    aux_v[pl.ds(64, 16)] = tot1

    @pl.when(wid == 0)
    def _():
        pltpu.sync_copy(aux_v.at[pl.ds(0, 48)], offs_hbm)
        pltpu.sync_copy(aux_v.at[pl.ds(48, 32)], tots_hbm)

    postbl[pl.ds(0, 16)] = off0 + pri0
    postbl[pl.ds(16, 16)] = off1 + pri1

    @pl.loop(0, N_CHUNKS)
    def _(ci):
        r0 = base_row + ci * CH_ROWS
        pltpu.sync_copy(src_hbm.at[pl.ds(r0, CH_ROWS)], sbuf)
        pltpu.sync_copy(dst_hbm.at[pl.ds(r0, CH_ROWS)], dbuf)
        for j in range(CH_ROWS):
            @pl.loop(0, 128, step=16)
            def _(k):
                sl = pl.ds(k, 16)
                s16 = sbuf[j, sl]
                d16 = dbuf[j, sl]
                o16 = _owner16(d16)
                dl16 = d16 - o16 * OWN
                pk16 = jnp.bitwise_or(s16, lax.shift_left(dl16, 17))
                c16, m16 = plsc.scan_count(o16)
                base16 = plsc.load_gather(postbl, [o16])
                pkbuf[j, sl] = pk16
                pos16 = base16 + (c16 - 1)
                pos16 = jnp.minimum(jnp.maximum(pos16, 0), EALLOC - 1)
                posbuf[j, sl] = pos16
                plsc.addupdate_scatter(postbl, [o16], c16, mask=m16)
        descs = [pltpu.async_copy(pkbuf.at[j], bpk_hbm.at[posbuf.at[j]], sem)
                 for j in range(CH_ROWS)]
        for d in descs:
            d.wait()


def _b2b_kernel(pot_hbm, bpk2_hbm, offs_hbm, tots_hbm, zer_hbm,
                bw2_hbm, ws0_hbm, ws1_hbm,
                node_v, pkbuf, wbuf, wadd, srcb, meta_v, shared,
                sem, wsem):
    cid = lax.axis_index("c")
    sid = lax.axis_index("s")
    wid = sid * 2 + cid

    @pl.when(sid == 0)
    def _():
        pltpu.sync_copy(zer_hbm, shared)
    pltpu.async_copy(pot_hbm, node_v, wsem).wait()
    pltpu.sync_copy(offs_hbm, meta_v.at[pl.ds(0, 48)])
    pltpu.sync_copy(tots_hbm, meta_v.at[pl.ds(48, 32)])
    plsc.subcore_barrier()

    offv = [meta_v[pl.ds(h * 16, 16)] for h in range(3)]
    totv = [meta_v[pl.ds(48 + h * 16, 16)] for h in range(2)]
    off_t = _pick(offv, wid)
    tot_t = _pick(totv, wid)
    row_t = lax.shift_right_logical(off_t, 7)
    row_t = jnp.minimum(jnp.maximum(row_t, 0), (EALLOC - CHUNK) // 128)
    nch = lax.shift_right_logical(_pick(offv, wid + 1) - off_t, 11)
    nch = jnp.minimum(jnp.maximum(nch, 0), EALLOC // CHUNK)
    iota16 = lax.iota(jnp.int32, 16)

    def chunk_body(ci, _):
        row0 = pl.multiple_of(row_t + ci * CH_ROWS, 16)
        pltpu.sync_copy(bpk2_hbm.at[pl.ds(row0, CH_ROWS)], pkbuf)
        for j in range(CH_ROWS):
            @pl.loop(0, 128, step=16)
            def _(k):
                sl = pl.ds(k, 16)
                pk = pkbuf[j, sl]
                s16 = jnp.bitwise_and(pk, 0x1FFFF)
                s16 = jnp.minimum(s16, NN - 1)
                d16 = wid * OWN + lax.shift_right_logical(pk, 17)
                d16 = jnp.minimum(d16, NN - 1)
                ps = plsc.load_gather(node_v, [s16])
                pd = plsc.load_gather(node_v, [d16])
                w16 = jnp.maximum(ps - pd, 0.0)
                wbuf[j, sl] = w16
                valid = (ci * CHUNK + j * 128 + k + iota16) < tot_t
                srcb[j, sl] = jnp.where(valid, s16, 0)
                wadd[j, sl] = jnp.where(valid, w16, 0.0)
        pltpu.sync_copy(wbuf, bw2_hbm.at[pl.ds(row0, CH_ROWS)])
        descs = [pltpu.async_copy(wadd.at[j], shared.at[srcb.at[j]], sem,
                                  add=True)
                 for j in range(CH_ROWS)]
        for d in descs:
            d.wait()
        return 0

    lax.fori_loop(0, nch, chunk_body, 0)

    plsc.subcore_barrier()

    @pl.when(sid == 0)
    def _():
        @pl.when(cid == 0)
        def _():
            pltpu.sync_copy(shared, ws0_hbm)

        @pl.when(cid == 1)
        def _():
            pltpu.sync_copy(shared, ws1_hbm)


def _iter2_kernel(g_hbm, bpk2_hbm, bw2_hbm, offs_hbm, tots_hbm,
                  melt_hbm, wst_hbm,
                  gn_hbm, dis_hbm,
                  node_v, pkbuf, wbuf, acc_v, slc_v, meta_v, wsem):
    cid = lax.axis_index("c")
    sid = lax.axis_index("s")
    wid = sid * 2 + cid

    cp = pltpu.async_copy(g_hbm, node_v, wsem)
    pltpu.sync_copy(offs_hbm, meta_v.at[pl.ds(0, 48)])
    pltpu.sync_copy(tots_hbm, meta_v.at[pl.ds(48, 32)])

    @pl.loop(0, OWN, step=16)
    def _(k):
        acc_v[pl.ds(k, 16)] = jnp.zeros((16,), jnp.float32)

    cp.wait()

    offv = [meta_v[pl.ds(h * 16, 16)] for h in range(3)]
    totv = [meta_v[pl.ds(48 + h * 16, 16)] for h in range(2)]
    off_t = _pick(offv, wid)
    tot_t = _pick(totv, wid)
    row_t = lax.shift_right_logical(off_t, 7)
    row_t = jnp.minimum(jnp.maximum(row_t, 0), (EALLOC - CHUNK) // 128)
    nfull = lax.shift_right_logical(tot_t, 11)
    nfull = jnp.minimum(jnp.maximum(nfull, 0), EALLOC // CHUNK - 1)
    rem = tot_t - lax.shift_left(nfull, 11)
    iota16 = lax.iota(jnp.int32, 16)

    def chunk_body(ci, masked):
        row0 = pl.multiple_of(row_t + ci * CH_ROWS, 16)
        pltpu.sync_copy(bpk2_hbm.at[pl.ds(row0, CH_ROWS)], pkbuf)
        pltpu.sync_copy(bw2_hbm.at[pl.ds(row0, CH_ROWS)], wbuf)
        for j in range(CH_ROWS):
            @pl.loop(0, 128, step=16)
            def _(k):
                sl = pl.ds(k, 16)
                pk = pkbuf[j, sl]
                s16 = jnp.bitwise_and(pk, 0x1FFFF)
                dl16 = lax.shift_right_logical(pk, 17)
                if masked:
                    s16 = jnp.minimum(s16, NN - 1)
                    dl16 = jnp.minimum(dl16, OWN - 1)
                gv = plsc.load_gather(node_v, [s16])
                v16 = gv * wbuf[j, sl]
                if masked:
                    valid = (j * 128 + k + iota16) < rem
                    v16 = jnp.where(valid, v16, 0.0)
                plsc.addupdate_scatter(acc_v, [dl16], v16)

    def full_body(ci, _):
        chunk_body(ci, False)
        return 0

    lax.fori_loop(0, nfull, full_body, 0)

    @pl.when(rem > 0)
    def _():
        chunk_body(nfull, True)

    nbase = pl.multiple_of(wid * OWN, 64)
    pltpu.sync_copy(melt_hbm.at[pl.ds(nbase, OWN)], slc_v)

    @pl.loop(0, OWN, step=16)
    def _(k):
        sl = pl.ds(k, 16)
        acc_v[sl] = acc_v[sl] + slc_v[sl]

    pltpu.sync_copy(acc_v, dis_hbm.at[pl.ds(nbase, OWN)])
    pltpu.sync_copy(wst_hbm.at[pl.ds(nbase, OWN)], slc_v)

    @pl.loop(0, OWN, step=16)
    def _(k):
        sl = pl.ds(k, 16)
        acc_v[sl] = acc_v[sl] / (slc_v[sl] + 1e-12)

    pltpu.sync_copy(acc_v, gn_hbm.at[pl.ds(nbase, OWN)])


def kernel(conduit_area, melt_rate, cell_area, bedrock_elevation,
           water_pressure, surface_slope, bedrock_slope,
           status_at_node, edge_index):
    f32 = jnp.float32
    npad = NPAD - NN
    ca = jnp.pad(conduit_area, (0, npad))
    mr = jnp.pad(melt_rate, (0, npad))
    cla = jnp.pad(cell_area, (0, npad))
    be = jnp.pad(bedrock_elevation, (0, npad))
    wp = jnp.pad(water_pressure, (0, npad))
    ss = jnp.pad(surface_slope, (0, npad))
    bs = jnp.pad(bedrock_slope, (0, npad))
    st = jnp.pad(status_at_node, (0, npad))

    epad = EPAD - EE
    pad_ids = (jnp.arange(epad, dtype=jnp.int32) * 61) % NN
    src = jnp.concatenate([edge_index[0], pad_ids]).reshape(EROWS, 128)
    dst = jnp.concatenate([edge_index[1], pad_ids]).reshape(EROWS, 128)
    zer = jnp.zeros((NPAD,), f32)

    melt, pot, geo = _tc_elementwise(_pre_body, 3, mr, cla, be, wp, ss, bs)

    hist_fn = pl.kernel(
        _hist_kernel,
        out_type=_i32((NWORK * NWORK,)),
        mesh=_SC_MESH,
        compiler_params=_SC_PARAMS,
        scratch_types=[
            pltpu.VMEM((CH_ROWS, 128), jnp.int32),
            pltpu.VMEM((NWORK,), jnp.int32),
            pltpu.SemaphoreType.DMA,
        ],
    )
    counts = hist_fn(dst)

    b2_fn = pl.kernel(
        _b2_kernel,
        out_type=[_i32((EALLOC,)), _i32((48,)), _i32((32,))],
        mesh=_SC_MESH,
        compiler_params=_SC_PARAMS,
        scratch_types=[
            pltpu.VMEM((CH_ROWS, 128), jnp.int32),
            pltpu.VMEM((CH_ROWS, 128), jnp.int32),
            pltpu.VMEM((CH_ROWS, 128), jnp.int32),
            pltpu.VMEM((CH_ROWS, 128), jnp.int32),
            pltpu.VMEM((NWORK * NWORK,), jnp.int32),
            pltpu.VMEM((NWORK,), jnp.int32),
            pltpu.VMEM((80,), jnp.int32),
            pltpu.SemaphoreType.DMA,
        ],
    )
    bpk, offs, tots = b2_fn(src, dst, counts)
    bpk2 = bpk.reshape(EALLOC // 128, 128)

    b2b_fn = pl.kernel(
        _b2b_kernel,
        out_type=[_f32((EALLOC // 128, 128)), _f32((NPAD,)), _f32((NPAD,))],
        mesh=_SC_MESH,
        compiler_params=_SC_PARAMS,
        scratch_types=[
            pltpu.VMEM((NPAD,), jnp.float32),
            pltpu.VMEM((CH_ROWS, 128), jnp.int32),
            pltpu.VMEM((CH_ROWS, 128), jnp.float32),
            pltpu.VMEM((CH_ROWS, 128), jnp.float32),
            pltpu.VMEM((CH_ROWS, 128), jnp.int32),
            pltpu.VMEM((80,), jnp.int32),
            pltpu.VMEM_SHARED((NPAD,), jnp.float32),
            pltpu.SemaphoreType.DMA,
            pltpu.SemaphoreType.DMA,
        ],
    )
    bw2, ws0, ws1 = b2b_fn(pot, bpk2, offs, tots, zer)

    g, wst = _tc_elementwise(_g0_body, 2, melt, ws0, ws1)

    iter_fn = pl.kernel(
        _iter2_kernel,
        out_type=[_f32((NPAD,)), _f32((NPAD,))],
        mesh=_SC_MESH,
        compiler_params=_SC_PARAMS,
        scratch_types=[
            pltpu.VMEM((NPAD,), jnp.float32),
            pltpu.VMEM((CH_ROWS, 128), jnp.int32),
            pltpu.VMEM((CH_ROWS, 128), jnp.float32),
            pltpu.VMEM((OWN,), jnp.float32),
            pltpu.VMEM((OWN,), jnp.float32),
            pltpu.VMEM((80,), jnp.int32),
            pltpu.SemaphoreType.DMA,
        ],
    )

    dis = melt
    for _ in range(N_FLOW_ITERS):
        g, dis = iter_fn(g, bpk2, bw2, offs, tots, melt, wst)

    hg = _tc_elementwise(_fin_body, 1, dis, ca, st.astype(f32))[0]

    return (hg[:NN], dis[:NN], pot[:NN], geo[:NN])
